# baseline (speedup 1.0000x reference)
import functools

import jax
import jax.numpy as jnp
import numpy as np
from jax import lax
from jax.experimental import pallas as pl
from jax.experimental.pallas import tpu as pltpu
from jax.experimental.pallas import tpu_sc as plsc

N = 1_000_000
TABLE_T = 2_097_152
NCH = 15
NPAD = 1 << 20
NW = 32
PER_W = NPAD // NW
CHUNK = 8192
NCHUNKS = PER_W // CHUNK
GROUPS = CHUNK // 128

P2 = np.int32(np.uint32(2654435761).view(np.int32))
P3 = np.int32(805459861)
MASK = np.int32(TABLE_T - 1)


def _sc_gather(tab_flat, cx, cy, cz):
    mesh = plsc.VectorSubcoreMesh(core_axis_name="c", subcore_axis_name="s")

    @functools.partial(
        pl.kernel,
        mesh=mesh,
        out_type=(
            jax.ShapeDtypeStruct((NCH * NPAD // 128, 128), jnp.float32),
            jax.ShapeDtypeStruct((NW * 2 * 16,), jnp.float32),
        ),
        scratch_types=[
            pltpu.VMEM((CHUNK,), jnp.int32),
            pltpu.VMEM((CHUNK,), jnp.int32),
            pltpu.VMEM((CHUNK,), jnp.int32),
            pltpu.VMEM((GROUPS, 128), jnp.int32),
            pltpu.VMEM((GROUPS, 128), jnp.int32),
            pltpu.VMEM((GROUPS, 128), jnp.float32),
            pltpu.VMEM((16,), jnp.float32),
            pltpu.VMEM((16,), jnp.float32),
            pltpu.SemaphoreType.DMA,
        ],
    )
    def k(tab_hbm, cx_hbm, cy_hbm, cz_hbm, g_hbm, part_hbm, c0, c1, c2, idx,
          idxc, gbuf, acc, accsq, sem):
        wid = lax.axis_index("s") * 2 + lax.axis_index("c")
        base_w = wid * PER_W
        acc[...] = jnp.zeros((16,), jnp.float32)
        accsq[...] = jnp.zeros((16,), jnp.float32)

        @pl.loop(0, NCHUNKS)
        def _chunk(ci):
            base = pl.multiple_of(base_w + ci * CHUNK, CHUNK)
            pltpu.sync_copy(cx_hbm.at[pl.ds(base, CHUNK)], c0)
            pltpu.sync_copy(cy_hbm.at[pl.ds(base, CHUNK)], c1)
            pltpu.sync_copy(cz_hbm.at[pl.ds(base, CHUNK)], c2)

            @pl.loop(0, GROUPS)
            def _hash(g):
                @pl.loop(0, 128, step=16)
                def _hash16(j):
                    o = g * 128 + j
                    h = (
                        c0[pl.ds(o, 16)]
                        ^ (c1[pl.ds(o, 16)] * P2)
                        ^ (c2[pl.ds(o, 16)] * P3)
                    )
                    idx[g, pl.ds(j, 16)] = h & MASK

            @pl.loop(0, NCH)
            def _ch(ch):
                off = ch * TABLE_T

                @pl.loop(0, GROUPS)
                def _fire(g):
                    @pl.loop(0, 128, step=16)
                    def _ofs16(j):
                        idxc[g, pl.ds(j, 16)] = idx[g, pl.ds(j, 16)] + off

                    pltpu.async_copy(tab_hbm.at[idxc.at[g]], gbuf.at[g], sem)

                @pl.loop(0, GROUPS)
                def _drain(g):
                    pltpu.make_async_copy(
                        tab_hbm.at[idxc.at[g]], gbuf.at[g], sem
                    ).wait()

                @pl.when(ch < 3)
                def _stats():
                    @pl.loop(0, GROUPS)
                    def _st(g):
                        @pl.loop(0, 128, step=16)
                        def _st16(j):
                            x = gbuf[g, pl.ds(j, 16)]
                            m = jnp.where(base + g * 128 + j < N,
                                          jnp.float32(1.0), jnp.float32(0.0))
                            acc[...] += x * m
                            accsq[...] += x * x * m

                row0 = pl.multiple_of(ch * (NPAD // 128) + base // 128, 8)
                pltpu.sync_copy(gbuf, g_hbm.at[pl.ds(row0, GROUPS)])

        pltpu.sync_copy(acc, part_hbm.at[pl.ds(pl.multiple_of(wid * 32, 16), 16)])
        pltpu.sync_copy(
            accsq, part_hbm.at[pl.ds(pl.multiple_of(wid * 32 + 16, 16), 16)]
        )

    return k(tab_flat, cx, cy, cz)


def _tc_act_body(g_ref, p_ref, fac_ref, o_ref):
    p = p_ref[...]
    fl = lax.broadcasted_iota(jnp.int32, p.shape, 0) * 128 + lax.broadcasted_iota(
        jnp.int32, p.shape, 1
    )
    is_sum = (fl % 32) < 16
    S = jnp.sum(jnp.where(is_sum, p, 0.0))
    SS = jnp.sum(jnp.where(is_sum, 0.0, p))
    M = jnp.float32(3 * N)
    mu = S / M
    var = (SS - S * S / M) / (M - 1.0)
    inv_sd = lax.rsqrt(var)
    f = fac_ref[0, 0]
    vs = fac_ref[0, 1]
    s1 = 2.0 * f / vs
    g = g_ref[...]
    rows = lax.broadcasted_iota(jnp.int32, g.shape, 0)
    sig = 1.0 / (1.0 + jnp.exp(-g))
    sig4 = 1.0 / (1.0 + jnp.exp(-(g - 4.0)))
    dm = (g - mu) * inv_sd * (s1 / 6.0)
    o_ref[...] = jnp.where(
        rows < 3,
        dm,
        jnp.where(
            rows < 7,
            g,
            jnp.where(rows < 10, sig * s1, jnp.where(rows == 13, sig4, sig)),
        ),
    )


def _tc_activate(g_raw, partials, fac):
    BN = 8192
    grid = (pl.cdiv(N, BN),)
    return pl.pallas_call(
        _tc_act_body,
        grid=grid,
        in_specs=[
            pl.BlockSpec((NCH, BN), lambda i: (0, i)),
            pl.BlockSpec((8, 128), lambda i: (0, 0)),
            pl.BlockSpec((1, 2), lambda i: (0, 0)),
        ],
        out_specs=pl.BlockSpec((NCH, BN), lambda i: (0, i)),
        out_shape=jax.ShapeDtypeStruct((NCH, N), jnp.float32),
    )(g_raw, partials, fac)


def kernel(hash_table, coordinates, far, voxel_size):
    ct = jnp.zeros((3, NPAD), jnp.int32).at[:, :N].set(coordinates.T)
    g_raw, partials = _sc_gather(hash_table.reshape(-1), ct[0], ct[1], ct[2])
    fac = jnp.stack(
        [far[0].astype(jnp.float32),
         jnp.asarray(voxel_size, jnp.float32)]
    ).reshape(1, 2)
    return _tc_activate(
        g_raw.reshape(NCH, NPAD), partials.reshape(8, 128), fac
    )
